# baseline (speedup 1.0000x reference)
import functools

import jax
import jax.numpy as jnp
from jax import lax
from jax.experimental import pallas as pl
from jax.experimental.pallas import tpu as pltpu
from jax.experimental.pallas import tpu_sc as plsc

NC, NS, L = 2, 16, 16
NW = NC * NS
N_PIX = 8 * 224 * 224
PER_W = N_PIX // NW
CHUNK = 128
NCHUNK = PER_W // CHUNK
TBL_ROWS = 512 * 512
D = 64
OUTC = 68

_ATAN_C = (0.15915440747490797, -0.05302772555124891, 0.03153370422192871,
           -0.021084069699430396, 0.012702314650757687,
           -0.005367620312675214, 0.0010890276221740287)


def _cell_coord(y, x):
    ax = jnp.abs(x)
    ay = jnp.abs(y)
    m = jnp.minimum(ax, ay)
    big = jnp.maximum(ax, ay)
    t = m / jnp.maximum(big, 1e-30)
    t2 = t * t
    p = jnp.float32(_ATAN_C[6])
    for c in _ATAN_C[5::-1]:
        p = p * t2 + jnp.float32(c)
    p = p * t
    r = jnp.where(ay > ax, 0.25 - p, p)
    r = jnp.where(x < 0.0, 0.5 - r, r)
    r = jnp.where(y < 0.0, 1.0 - r, r)
    return r * 511.0


def _sc_body(img_hbm, tab_hbm, out_hbm,
             img_v, w00, w10, w01, w11, i00, i10, i01, i11,
             g00, g10, g01, g11, out_v, sem):
    wid = lax.axis_index("s") * NC + lax.axis_index("c")
    base0 = wid * PER_W
    lane = lax.broadcasted_iota(jnp.int32, (L,), 0)
    zero = jnp.zeros((L,), jnp.int32)

    def do_chunk(ci, carry):
        base = base0 + ci * CHUNK
        pltpu.sync_copy(img_hbm.at[pl.ds(base * 4, CHUNK * 4)], img_v)

        def group(g, c2):
            bi = g * L
            rows = bi + lane
            flat4 = rows * 4
            x0 = plsc.load_gather(img_v, [flat4])
            y0 = plsc.load_gather(img_v, [flat4 + 1])
            x1 = plsc.load_gather(img_v, [flat4 + 2])
            y1 = plsc.load_gather(img_v, [flat4 + 3])
            c0 = jnp.minimum(jnp.maximum(_cell_coord(y0, x0), 0.0), 510.0)
            c1 = jnp.minimum(jnp.maximum(_cell_coord(y1, x1), 0.0), 510.0)
            xi0 = c0.astype(jnp.int32)
            xi1 = c1.astype(jnp.int32)
            f0 = c0 - xi0.astype(jnp.float32)
            f1 = c1 - xi1.astype(jnp.float32)
            r = xi0 * 512 + xi1
            i00[pl.ds(bi, L)] = r
            i10[pl.ds(bi, L)] = r + 512
            i01[pl.ds(bi, L)] = r + 1
            i11[pl.ds(bi, L)] = r + 513
            w00[pl.ds(bi, L)] = (1.0 - f0) * (1.0 - f1)
            w10[pl.ds(bi, L)] = f0 * (1.0 - f1)
            w01[pl.ds(bi, L)] = (1.0 - f0) * f1
            w11[pl.ds(bi, L)] = f0 * f1
            flat_o = rows * OUTC
            plsc.store_scatter(out_v, [flat_o + 64], x0)
            plsc.store_scatter(out_v, [flat_o + 65], y0)
            plsc.store_scatter(out_v, [flat_o + 66], x1)
            plsc.store_scatter(out_v, [flat_o + 67], y1)
            return c2

        lax.fori_loop(0, CHUNK // L, group, 0)

        d0 = pltpu.async_copy(tab_hbm.at[i00], g00, sem)
        d1 = pltpu.async_copy(tab_hbm.at[i10], g10, sem)
        d2 = pltpu.async_copy(tab_hbm.at[i01], g01, sem)
        d3 = pltpu.async_copy(tab_hbm.at[i11], g11, sem)
        d0.wait()
        d1.wait()
        d2.wait()
        d3.wait()

        def pix(p, c2):
            s00 = plsc.load_gather(w00, [zero + p])
            s10 = plsc.load_gather(w10, [zero + p])
            s01 = plsc.load_gather(w01, [zero + p])
            s11 = plsc.load_gather(w11, [zero + p])
            for q in range(D // L):
                sl = pl.ds(q * L, L)
                acc = (g00[p, sl] * s00 + g10[p, sl] * s10
                       + g01[p, sl] * s01 + g11[p, sl] * s11)
                out_v[pl.ds(p * OUTC + q * L, L)] = acc * 0.0005
            return c2

        lax.fori_loop(0, CHUNK, pix, 0)
        pltpu.sync_copy(out_v, out_hbm.at[pl.ds(base * OUTC, CHUNK * OUTC)])
        return carry

    lax.fori_loop(0, NCHUNK, do_chunk, 0)


_warp_sc = functools.partial(
    pl.kernel,
    out_type=jax.ShapeDtypeStruct((N_PIX * OUTC,), jnp.float32),
    mesh=plsc.VectorSubcoreMesh(core_axis_name="c", subcore_axis_name="s"),
    compiler_params=pltpu.CompilerParams(needs_layout_passes=False,
                                         use_tc_tiling_on_sc=False),
    scratch_types=[
        pltpu.VMEM((CHUNK * 4,), jnp.float32),
        pltpu.VMEM((CHUNK,), jnp.float32),
        pltpu.VMEM((CHUNK,), jnp.float32),
        pltpu.VMEM((CHUNK,), jnp.float32),
        pltpu.VMEM((CHUNK,), jnp.float32),
        pltpu.VMEM((CHUNK,), jnp.int32),
        pltpu.VMEM((CHUNK,), jnp.int32),
        pltpu.VMEM((CHUNK,), jnp.int32),
        pltpu.VMEM((CHUNK,), jnp.int32),
        pltpu.VMEM((CHUNK, D), jnp.float32),
        pltpu.VMEM((CHUNK, D), jnp.float32),
        pltpu.VMEM((CHUNK, D), jnp.float32),
        pltpu.VMEM((CHUNK, D), jnp.float32),
        pltpu.VMEM((CHUNK * OUTC,), jnp.float32),
        pltpu.SemaphoreType.DMA,
    ],
)(_sc_body)


def kernel(image, weight):
    img2 = image.reshape(N_PIX * 4)
    tab2 = weight.reshape(TBL_ROWS, D)
    out = _warp_sc(img2, tab2)
    return out.reshape(image.shape[0], image.shape[1], image.shape[2], OUTC)
